# Initial kernel scaffold; baseline (speedup 1.0000x reference)
import jax
import jax.numpy as jnp
from jax.experimental import pallas as pl


def kernel(edge_index, node_features, edge_features, W1, b1, W2, b2):
    raise NotImplementedError("write your pallas kernel here")


# speedup vs baseline: 2.0083x; 2.0083x over previous
import functools

import jax
import jax.numpy as jnp
from jax import lax
from jax.experimental import pallas as pl
from jax.experimental.pallas import tpu as pltpu
from jax.experimental.pallas import tpu_sc as plsc

N_NODES = 10000
N_EDGES = 320000
D_NODE = 128
D_EDGE = 16
D_HID = 64
D_OUT = 16

_NODE_BLK = 1000


def _proj_body(nf_ref, ws_ref, wt_ref, ts_ref, tt_ref):
    nf = nf_ref[...]
    ts_ref[...] = jnp.dot(nf, ws_ref[...], preferred_element_type=jnp.float32)
    tt_ref[...] = jnp.dot(nf, wt_ref[...], preferred_element_type=jnp.float32)


_proj_call = pl.pallas_call(
    _proj_body,
    grid=(N_NODES // _NODE_BLK,),
    in_specs=[
        pl.BlockSpec((_NODE_BLK, D_NODE), lambda i: (i, 0)),
        pl.BlockSpec((D_NODE, D_HID), lambda i: (0, 0)),
        pl.BlockSpec((D_NODE, D_HID), lambda i: (0, 0)),
    ],
    out_specs=[
        pl.BlockSpec((_NODE_BLK, D_HID), lambda i: (i, 0)),
        pl.BlockSpec((_NODE_BLK, D_HID), lambda i: (i, 0)),
    ],
    out_shape=[
        jax.ShapeDtypeStruct((N_NODES, D_HID), jnp.float32),
        jax.ShapeDtypeStruct((N_NODES, D_HID), jnp.float32),
    ],
)

_NC = 2
_NS = 16
_NW = _NC * _NS
_EPW = N_EDGES // _NW
_CHUNK = 80
_ITERS = _EPW // _CHUNK

@functools.cache
def _make_sc_gather():
    mesh = plsc.VectorSubcoreMesh(core_axis_name="c", subcore_axis_name="s")

    @functools.partial(
        pl.kernel,
        mesh=mesh,
        compiler_params=pltpu.CompilerParams(use_tc_tiling_on_sc=False),
        out_type=[
            jax.ShapeDtypeStruct((N_EDGES, D_HID), jnp.float32),
            jax.ShapeDtypeStruct((N_EDGES, D_HID), jnp.float32),
        ],
        scratch_types=[
            pltpu.VMEM((_CHUNK,), jnp.int32),
            pltpu.VMEM((_CHUNK,), jnp.int32),
            pltpu.VMEM((_CHUNK, D_HID), jnp.float32),
            pltpu.VMEM((_CHUNK, D_HID), jnp.float32),
            pltpu.SemaphoreType.DMA,
        ],
    )
    def _sc_gather(src_hbm, tgt_hbm, ts_hbm, tt_hbm, ps_hbm, pt_hbm,
                   idx_s, idx_t, rows_s, rows_t, sem):
        wid = lax.axis_index("s") * _NC + lax.axis_index("c")
        base = wid * _EPW

        def body(i, carry):
            off = base + i * _CHUNK
            pltpu.sync_copy(src_hbm.at[pl.ds(off, _CHUNK)], idx_s)
            pltpu.sync_copy(tgt_hbm.at[pl.ds(off, _CHUNK)], idx_t)
            cp1 = pltpu.async_copy(ts_hbm.at[idx_s], rows_s, sem)
            cp2 = pltpu.async_copy(tt_hbm.at[idx_t], rows_t, sem)
            cp1.wait()
            cp2.wait()
            pltpu.sync_copy(rows_s, ps_hbm.at[pl.ds(off, _CHUNK)])
            pltpu.sync_copy(rows_t, pt_hbm.at[pl.ds(off, _CHUNK)])
            return carry

        lax.fori_loop(0, _ITERS, body, 0)

    return _sc_gather


_EDGE_BLK = 8000


def _mlp_body(ef_ref, ps_ref, pt_ref, w1e_ref, b1_ref, w2_ref, b2_ref, out_ref):
    h = jnp.dot(ef_ref[...], w1e_ref[...], preferred_element_type=jnp.float32)
    h = h + ps_ref[...] + pt_ref[...] + b1_ref[...]
    h = jnp.maximum(h, 0.0)
    out_ref[...] = (
        jnp.dot(h, w2_ref[...], preferred_element_type=jnp.float32) + b2_ref[...]
    )


_mlp_call = pl.pallas_call(
    _mlp_body,
    grid=(N_EDGES // _EDGE_BLK,),
    in_specs=[
        pl.BlockSpec((_EDGE_BLK, D_EDGE), lambda i: (i, 0)),
        pl.BlockSpec((_EDGE_BLK, D_HID), lambda i: (i, 0)),
        pl.BlockSpec((_EDGE_BLK, D_HID), lambda i: (i, 0)),
        pl.BlockSpec((D_EDGE, D_HID), lambda i: (0, 0)),
        pl.BlockSpec((1, D_HID), lambda i: (0, 0)),
        pl.BlockSpec((D_HID, D_OUT), lambda i: (0, 0)),
        pl.BlockSpec((1, D_OUT), lambda i: (0, 0)),
    ],
    out_specs=pl.BlockSpec((_EDGE_BLK, D_OUT), lambda i: (i, 0)),
    out_shape=jax.ShapeDtypeStruct((N_EDGES, D_OUT), jnp.float32),
)


def kernel(edge_index, node_features, edge_features, W1, b1, W2, b2):
    src = edge_index[0].astype(jnp.int32)
    tgt = edge_index[1].astype(jnp.int32)
    w1e = W1[:D_EDGE]
    w1s = W1[D_EDGE:D_EDGE + D_NODE]
    w1t = W1[D_EDGE + D_NODE:]
    ts, tt = _proj_call(node_features, w1s, w1t)
    ps, pt = _make_sc_gather()(src, tgt, ts, tt)
    return _mlp_call(edge_features, ps, pt, w1e,
                     b1.reshape(1, D_HID), W2, b2.reshape(1, D_OUT))


# speedup vs baseline: 2.0836x; 1.0375x over previous
import functools

import jax
import jax.numpy as jnp
from jax import lax
from jax.experimental import pallas as pl
from jax.experimental.pallas import tpu as pltpu
from jax.experimental.pallas import tpu_sc as plsc

N_NODES = 10000
N_EDGES = 320000
D_NODE = 128
D_EDGE = 16
D_HID = 64
D_OUT = 16

_NODE_BLK = 1000


def _proj_body(nf_ref, ws_ref, wt_ref, ts_ref, tt_ref):
    nf = nf_ref[...]
    ts_ref[...] = jnp.dot(nf, ws_ref[...], preferred_element_type=jnp.float32)
    tt_ref[...] = jnp.dot(nf, wt_ref[...], preferred_element_type=jnp.float32)


_proj_call = pl.pallas_call(
    _proj_body,
    grid=(N_NODES // _NODE_BLK,),
    in_specs=[
        pl.BlockSpec((_NODE_BLK, D_NODE), lambda i: (i, 0)),
        pl.BlockSpec((D_NODE, D_HID), lambda i: (0, 0)),
        pl.BlockSpec((D_NODE, D_HID), lambda i: (0, 0)),
    ],
    out_specs=[
        pl.BlockSpec((_NODE_BLK, D_HID), lambda i: (i, 0)),
        pl.BlockSpec((_NODE_BLK, D_HID), lambda i: (i, 0)),
    ],
    out_shape=[
        jax.ShapeDtypeStruct((N_NODES, D_HID), jnp.float32),
        jax.ShapeDtypeStruct((N_NODES, D_HID), jnp.float32),
    ],
)

_NC = 2
_NS = 16
_NW = _NC * _NS
_EPW = N_EDGES // _NW
_CHUNK = 80
_ITERS = _EPW // _CHUNK
_L = 16


@functools.cache
def _make_sc_gather():
    mesh = plsc.VectorSubcoreMesh(core_axis_name="c", subcore_axis_name="s")

    @functools.partial(
        pl.kernel,
        mesh=mesh,
        compiler_params=pltpu.CompilerParams(use_tc_tiling_on_sc=False),
        out_type=jax.ShapeDtypeStruct((N_EDGES // 2, 2 * D_HID), jnp.float32),
        scratch_types=[
            pltpu.VMEM((_EPW,), jnp.int32),
            pltpu.VMEM((_EPW,), jnp.int32),
            pltpu.VMEM((_CHUNK, D_HID), jnp.float32),
            pltpu.VMEM((_CHUNK, D_HID), jnp.float32),
            pltpu.VMEM((_CHUNK // 2, 2 * D_HID), jnp.float32),
            pltpu.SemaphoreType.DMA,
        ],
    )
    def _sc_gather(src_hbm, tgt_hbm, ts_hbm, tt_hbm, pres_hbm,
                   idx_s, idx_t, rows_s, rows_t, packed, sem):
        wid = lax.axis_index("s") * _NC + lax.axis_index("c")
        base = wid * _EPW
        pltpu.sync_copy(src_hbm.at[pl.ds(base, _EPW)], idx_s)
        pltpu.sync_copy(tgt_hbm.at[pl.ds(base, _EPW)], idx_t)

        def body(i, carry):
            off = i * _CHUNK
            cp1 = pltpu.async_copy(
                ts_hbm.at[idx_s.at[pl.ds(off, _CHUNK)]], rows_s, sem)
            cp2 = pltpu.async_copy(
                tt_hbm.at[idx_t.at[pl.ds(off, _CHUNK)]], rows_t, sem)
            cp1.wait()
            cp2.wait()
            for p in range(_CHUNK // 2):
                for c in range(D_HID // _L):
                    out_lo = pl.ds(c * _L, _L)
                    out_hi = pl.ds(D_HID + c * _L, _L)
                    packed[p, out_lo] = (
                        rows_s[2 * p, pl.ds(c * _L, _L)]
                        + rows_t[2 * p, pl.ds(c * _L, _L)])
                    packed[p, out_hi] = (
                        rows_s[2 * p + 1, pl.ds(c * _L, _L)]
                        + rows_t[2 * p + 1, pl.ds(c * _L, _L)])
            pltpu.sync_copy(
                packed, pres_hbm.at[pl.ds((base + off) // 2, _CHUNK // 2)])
            return carry

        lax.fori_loop(0, _ITERS, body, 0)

    return _sc_gather


_PAIR_BLK = 6400
_N_PAIRS = N_EDGES // 2


def _mlp_body(ef2t_ref, pres_ref, w1e2_ref, b1_ref, w2_ref, b2t_ref, out_ref):
    contrib = lax.dot_general(
        ef2t_ref[...], w1e2_ref[...], (((0,), (0,)), ((), ())),
        preferred_element_type=jnp.float32)
    h = jnp.maximum(contrib + pres_ref[...] + b1_ref[...], 0.0)
    out_ref[...] = lax.dot_general(
        w2_ref[...], h, (((0,), (1,)), ((), ())),
        preferred_element_type=jnp.float32) + b2t_ref[...]


_mlp_call = pl.pallas_call(
    _mlp_body,
    grid=(_N_PAIRS // _PAIR_BLK,),
    in_specs=[
        pl.BlockSpec((2 * D_EDGE, _PAIR_BLK), lambda i: (0, i)),
        pl.BlockSpec((_PAIR_BLK, 2 * D_HID), lambda i: (i, 0)),
        pl.BlockSpec((2 * D_EDGE, 2 * D_HID), lambda i: (0, 0)),
        pl.BlockSpec((1, 2 * D_HID), lambda i: (0, 0)),
        pl.BlockSpec((2 * D_HID, 2 * D_OUT), lambda i: (0, 0)),
        pl.BlockSpec((2 * D_OUT, 1), lambda i: (0, 0)),
    ],
    out_specs=pl.BlockSpec((2 * D_OUT, _PAIR_BLK), lambda i: (0, i)),
    out_shape=jax.ShapeDtypeStruct((2 * D_OUT, _N_PAIRS), jnp.float32),
)


def _block_diag2(w):
    z = jnp.zeros_like(w)
    return jnp.concatenate(
        [jnp.concatenate([w, z], axis=1), jnp.concatenate([z, w], axis=1)],
        axis=0)


def kernel(edge_index, node_features, edge_features, W1, b1, W2, b2):
    src = edge_index[0].astype(jnp.int32)
    tgt = edge_index[1].astype(jnp.int32)
    w1e = W1[:D_EDGE]
    w1s = W1[D_EDGE:D_EDGE + D_NODE]
    w1t = W1[D_EDGE + D_NODE:]
    ts, tt = _proj_call(node_features, w1s, w1t)
    presum2 = _make_sc_gather()(src, tgt, ts, tt)
    eft = jnp.transpose(edge_features)
    ef2t = jnp.concatenate([eft[:, 0::2], eft[:, 1::2]], axis=0)
    w1e2 = _block_diag2(w1e)
    w2_2 = _block_diag2(W2)
    b1_2 = jnp.concatenate([b1, b1]).reshape(1, 2 * D_HID)
    b2t = jnp.concatenate([b2, b2]).reshape(2 * D_OUT, 1)
    out2t = _mlp_call(ef2t, presum2, w1e2, b1_2, w2_2, b2t)
    outt = jnp.stack([out2t[:D_OUT], out2t[D_OUT:]], axis=2)
    outt = outt.reshape(D_OUT, N_EDGES)
    return jnp.transpose(outt)


# speedup vs baseline: 4.4153x; 2.1190x over previous
import functools

import jax
import jax.numpy as jnp
from jax import lax
from jax.experimental import pallas as pl
from jax.experimental.pallas import tpu as pltpu
from jax.experimental.pallas import tpu_sc as plsc

N_NODES = 10000
N_EDGES = 320000
D_NODE = 128
D_EDGE = 16
D_HID = 64
D_OUT = 16
_HALF = N_EDGES // 2

_NODE_BLK = 1000


def _proj_body(nf_ref, ws_ref, wt_ref, ts_ref, tt_ref):
    nf = nf_ref[...]
    ts_ref[...] = jnp.dot(nf, ws_ref[...], preferred_element_type=jnp.float32)
    tt_ref[...] = jnp.dot(nf, wt_ref[...], preferred_element_type=jnp.float32)


_proj_call = pl.pallas_call(
    _proj_body,
    grid=(N_NODES // _NODE_BLK,),
    in_specs=[
        pl.BlockSpec((_NODE_BLK, D_NODE), lambda i: (i, 0)),
        pl.BlockSpec((D_NODE, D_HID), lambda i: (0, 0)),
        pl.BlockSpec((D_NODE, D_HID), lambda i: (0, 0)),
    ],
    out_specs=[
        pl.BlockSpec((_NODE_BLK, D_HID), lambda i: (i, 0)),
        pl.BlockSpec((_NODE_BLK, D_HID), lambda i: (i, 0)),
    ],
    out_shape=[
        jax.ShapeDtypeStruct((N_NODES, D_HID), jnp.float32),
        jax.ShapeDtypeStruct((N_NODES, D_HID), jnp.float32),
    ],
)

_NC = 2
_NS = 16
_NW = _NC * _NS
_PPW = _HALF // _NW
_PCHUNK = 40
_ITERS = _PPW // _PCHUNK
_L = 16


@functools.cache
def _make_sc_gather():
    mesh = plsc.VectorSubcoreMesh(core_axis_name="c", subcore_axis_name="s")

    @functools.partial(
        pl.kernel,
        mesh=mesh,
        compiler_params=pltpu.CompilerParams(use_tc_tiling_on_sc=False),
        out_type=jax.ShapeDtypeStruct((_HALF, 2 * D_HID), jnp.float32),
        scratch_types=[
            pltpu.VMEM((_PPW,), jnp.int32),
            pltpu.VMEM((_PPW,), jnp.int32),
            pltpu.VMEM((_PPW,), jnp.int32),
            pltpu.VMEM((_PPW,), jnp.int32),
            pltpu.VMEM((_PCHUNK, D_HID), jnp.float32),
            pltpu.VMEM((_PCHUNK, D_HID), jnp.float32),
            pltpu.VMEM((_PCHUNK, D_HID), jnp.float32),
            pltpu.VMEM((_PCHUNK, D_HID), jnp.float32),
            pltpu.VMEM((_PCHUNK, 2 * D_HID), jnp.float32),
            pltpu.SemaphoreType.DMA,
        ],
    )
    def _sc_gather(src_hbm, tgt_hbm, ts_hbm, tt_hbm, pres_hbm,
                   idx_s_lo, idx_t_lo, idx_s_hi, idx_t_hi,
                   rs_lo, rt_lo, rs_hi, rt_hi, packed, sem):
        wid = lax.axis_index("s") * _NC + lax.axis_index("c")
        base = wid * _PPW
        pltpu.sync_copy(src_hbm.at[pl.ds(base, _PPW)], idx_s_lo)
        pltpu.sync_copy(tgt_hbm.at[pl.ds(base, _PPW)], idx_t_lo)
        pltpu.sync_copy(src_hbm.at[pl.ds(_HALF + base, _PPW)], idx_s_hi)
        pltpu.sync_copy(tgt_hbm.at[pl.ds(_HALF + base, _PPW)], idx_t_hi)

        def body(i, carry):
            off = i * _PCHUNK
            sl = pl.ds(off, _PCHUNK)
            cp1 = pltpu.async_copy(ts_hbm.at[idx_s_lo.at[sl]], rs_lo, sem)
            cp2 = pltpu.async_copy(tt_hbm.at[idx_t_lo.at[sl]], rt_lo, sem)
            cp3 = pltpu.async_copy(ts_hbm.at[idx_s_hi.at[sl]], rs_hi, sem)
            cp4 = pltpu.async_copy(tt_hbm.at[idx_t_hi.at[sl]], rt_hi, sem)
            cp1.wait()
            cp2.wait()
            cp3.wait()
            cp4.wait()
            for p in range(_PCHUNK):
                for c in range(D_HID // _L):
                    ls = pl.ds(c * _L, _L)
                    packed[p, pl.ds(c * _L, _L)] = (
                        rs_lo[p, ls] + rt_lo[p, ls])
                    packed[p, pl.ds(D_HID + c * _L, _L)] = (
                        rs_hi[p, ls] + rt_hi[p, ls])
            pltpu.sync_copy(packed, pres_hbm.at[pl.ds(base + off, _PCHUNK)])
            return carry

        lax.fori_loop(0, _ITERS, body, 0)

    return _sc_gather


_PAIR_BLK = 6400
_N_BLKS = _HALF // _PAIR_BLK


def _mlp_body(eflo_ref, efhi_ref, pres_ref, w1e_ref, b1_ref, w2_ref, b2_ref,
              olo_ref, ohi_ref):
    pres = pres_ref[...]
    w1e = w1e_ref[...]
    b1 = b1_ref[...]
    w2 = w2_ref[...]
    b2 = b2_ref[...]
    dn_in = (((0,), (0,)), ((), ()))
    dn_out = (((0,), (1,)), ((), ()))
    clo = lax.dot_general(eflo_ref[...], w1e, dn_in,
                          preferred_element_type=jnp.float32)
    chi = lax.dot_general(efhi_ref[...], w1e, dn_in,
                          preferred_element_type=jnp.float32)
    hlo = jnp.maximum(clo + pres[:, :D_HID] + b1, 0.0)
    hhi = jnp.maximum(chi + pres[:, D_HID:] + b1, 0.0)
    olo_ref[...] = lax.dot_general(w2, hlo, dn_out,
                                   preferred_element_type=jnp.float32) + b2
    ohi_ref[...] = lax.dot_general(w2, hhi, dn_out,
                                   preferred_element_type=jnp.float32) + b2


_mlp_call = pl.pallas_call(
    _mlp_body,
    grid=(_N_BLKS,),
    in_specs=[
        pl.BlockSpec((D_EDGE, _PAIR_BLK), lambda i: (0, i)),
        pl.BlockSpec((D_EDGE, _PAIR_BLK), lambda i: (0, i + _N_BLKS)),
        pl.BlockSpec((_PAIR_BLK, 2 * D_HID), lambda i: (i, 0)),
        pl.BlockSpec((D_EDGE, D_HID), lambda i: (0, 0)),
        pl.BlockSpec((1, D_HID), lambda i: (0, 0)),
        pl.BlockSpec((D_HID, D_OUT), lambda i: (0, 0)),
        pl.BlockSpec((D_OUT, 1), lambda i: (0, 0)),
    ],
    out_specs=[
        pl.BlockSpec((D_OUT, _PAIR_BLK), lambda i: (0, i)),
        pl.BlockSpec((D_OUT, _PAIR_BLK), lambda i: (0, i)),
    ],
    out_shape=[
        jax.ShapeDtypeStruct((D_OUT, _HALF), jnp.float32),
        jax.ShapeDtypeStruct((D_OUT, _HALF), jnp.float32),
    ],
)


def kernel(edge_index, node_features, edge_features, W1, b1, W2, b2):
    src = edge_index[0].astype(jnp.int32)
    tgt = edge_index[1].astype(jnp.int32)
    w1e = W1[:D_EDGE]
    w1s = W1[D_EDGE:D_EDGE + D_NODE]
    w1t = W1[D_EDGE + D_NODE:]
    ts, tt = _proj_call(node_features, w1s, w1t)
    presum2 = _make_sc_gather()(src, tgt, ts, tt)
    eft = jnp.transpose(edge_features)
    out_lo, out_hi = _mlp_call(
        eft, eft, presum2, w1e, b1.reshape(1, D_HID), W2,
        b2.reshape(D_OUT, 1))
    outt = jnp.concatenate([out_lo, out_hi], axis=1)
    return jnp.transpose(outt)


# speedup vs baseline: 5.6152x; 1.2718x over previous
import functools

import jax
import jax.numpy as jnp
from jax import lax
from jax.experimental import pallas as pl
from jax.experimental.pallas import tpu as pltpu
from jax.experimental.pallas import tpu_sc as plsc

N_NODES = 10000
N_EDGES = 320000
D_NODE = 128
D_EDGE = 16
D_HID = 64
D_OUT = 16
_HALF = N_EDGES // 2

_NODE_BLK = 1000


def _proj_body(nf_ref, ws_ref, wt_ref, ts_ref, tt_ref):
    nf = nf_ref[...]
    ts_ref[...] = jnp.dot(nf, ws_ref[...], preferred_element_type=jnp.float32)
    tt_ref[...] = jnp.dot(nf, wt_ref[...], preferred_element_type=jnp.float32)


_proj_call = pl.pallas_call(
    _proj_body,
    grid=(N_NODES // _NODE_BLK,),
    in_specs=[
        pl.BlockSpec((_NODE_BLK, D_NODE), lambda i: (i, 0)),
        pl.BlockSpec((D_NODE, D_HID), lambda i: (0, 0)),
        pl.BlockSpec((D_NODE, D_HID), lambda i: (0, 0)),
    ],
    out_specs=[
        pl.BlockSpec((_NODE_BLK, D_HID), lambda i: (i, 0)),
        pl.BlockSpec((_NODE_BLK, D_HID), lambda i: (i, 0)),
    ],
    out_shape=[
        jax.ShapeDtypeStruct((N_NODES, D_HID), jnp.float32),
        jax.ShapeDtypeStruct((N_NODES, D_HID), jnp.float32),
    ],
)

_NC = 2
_NS = 16
_NW = _NC * _NS
_PPW = _HALF // _NW
_PCHUNK = 40
_ITERS = _PPW // _PCHUNK
_L = 16


@functools.cache
def _make_sc_gather():
    mesh = plsc.VectorSubcoreMesh(core_axis_name="c", subcore_axis_name="s")

    row_t = pltpu.VMEM((_PCHUNK, D_HID), jnp.float32)
    packed_t = pltpu.VMEM((_PCHUNK, 2 * D_HID), jnp.float32)

    @functools.partial(
        pl.kernel,
        mesh=mesh,
        compiler_params=pltpu.CompilerParams(use_tc_tiling_on_sc=False),
        out_type=jax.ShapeDtypeStruct((_HALF, 2 * D_HID), jnp.float32),
        scratch_types=[
            pltpu.VMEM((_PPW,), jnp.int32),
            pltpu.VMEM((_PPW,), jnp.int32),
            pltpu.VMEM((_PPW,), jnp.int32),
            pltpu.VMEM((_PPW,), jnp.int32),
            row_t, row_t, row_t, row_t,
            row_t, row_t, row_t, row_t,
            packed_t, packed_t,
            pltpu.SemaphoreType.DMA,
            pltpu.SemaphoreType.DMA,
            pltpu.SemaphoreType.DMA,
            pltpu.SemaphoreType.DMA,
        ],
    )
    def _sc_gather(src_hbm, tgt_hbm, ts_hbm, tt_hbm, pres_hbm,
                   idx_s_lo, idx_t_lo, idx_s_hi, idx_t_hi,
                   a0, a1, a2, a3, b0, b1_, b2_, b3, pk_a, pk_b,
                   sem_a, sem_b, st_a, st_b):
        wid = lax.axis_index("s") * _NC + lax.axis_index("c")
        base = wid * _PPW
        pltpu.sync_copy(src_hbm.at[pl.ds(base, _PPW)], idx_s_lo)
        pltpu.sync_copy(tgt_hbm.at[pl.ds(base, _PPW)], idx_t_lo)
        pltpu.sync_copy(src_hbm.at[pl.ds(_HALF + base, _PPW)], idx_s_hi)
        pltpu.sync_copy(tgt_hbm.at[pl.ds(_HALF + base, _PPW)], idx_t_hi)

        set_a = (a0, a1, a2, a3)
        set_b = (b0, b1_, b2_, b3)

        def fire(bufs, sem, i):
            sl = pl.ds(i * _PCHUNK, _PCHUNK)
            pltpu.async_copy(ts_hbm.at[idx_s_lo.at[sl]], bufs[0], sem)
            pltpu.async_copy(tt_hbm.at[idx_t_lo.at[sl]], bufs[1], sem)
            pltpu.async_copy(ts_hbm.at[idx_s_hi.at[sl]], bufs[2], sem)
            pltpu.async_copy(tt_hbm.at[idx_t_hi.at[sl]], bufs[3], sem)

        def wait_gathers(bufs, sem):
            sl = pl.ds(0, _PCHUNK)
            pltpu.make_async_copy(ts_hbm.at[idx_s_lo.at[sl]], bufs[0], sem).wait()
            pltpu.make_async_copy(tt_hbm.at[idx_t_lo.at[sl]], bufs[1], sem).wait()
            pltpu.make_async_copy(ts_hbm.at[idx_s_hi.at[sl]], bufs[2], sem).wait()
            pltpu.make_async_copy(tt_hbm.at[idx_t_hi.at[sl]], bufs[3], sem).wait()

        def add_pack(bufs, pk):
            rs_lo, rt_lo, rs_hi, rt_hi = bufs
            for p in range(_PCHUNK):
                for c in range(D_HID // _L):
                    ls = pl.ds(c * _L, _L)
                    pk[p, pl.ds(c * _L, _L)] = rs_lo[p, ls] + rt_lo[p, ls]
                    pk[p, pl.ds(D_HID + c * _L, _L)] = (
                        rs_hi[p, ls] + rt_hi[p, ls])

        def drain_store(pk, st):
            pltpu.make_async_copy(
                pk, pres_hbm.at[pl.ds(base, _PCHUNK)], st).wait()

        def store(pk, st, i):
            pltpu.async_copy(
                pk, pres_hbm.at[pl.ds(base + i * _PCHUNK, _PCHUNK)], st)

        fire(set_a, sem_a, 0)

        def body(j, carry):
            fire(set_b, sem_b, 2 * j + 1)
            @pl.when(j > 0)
            def _():
                drain_store(pk_a, st_a)
            wait_gathers(set_a, sem_a)
            add_pack(set_a, pk_a)
            store(pk_a, st_a, 2 * j)
            fire(set_a, sem_a, 2 * j + 2)
            @pl.when(j > 0)
            def _():
                drain_store(pk_b, st_b)
            wait_gathers(set_b, sem_b)
            add_pack(set_b, pk_b)
            store(pk_b, st_b, 2 * j + 1)
            return carry

        lax.fori_loop(0, (_ITERS - 1) // 2, body, 0)

        drain_store(pk_a, st_a)
        wait_gathers(set_a, sem_a)
        add_pack(set_a, pk_a)
        store(pk_a, st_a, _ITERS - 1)
        drain_store(pk_a, st_a)
        drain_store(pk_b, st_b)

    return _sc_gather


_PAIR_BLK = 6400
_N_BLKS = _HALF // _PAIR_BLK


def _mlp_body(eflo_ref, efhi_ref, pres_ref, w1e_ref, b1_ref, w2_ref, b2_ref,
              olo_ref, ohi_ref):
    pres = pres_ref[...]
    w1e = w1e_ref[...]
    b1 = b1_ref[...]
    w2 = w2_ref[...]
    b2 = b2_ref[...]
    dn_in = (((0,), (0,)), ((), ()))
    dn_out = (((0,), (1,)), ((), ()))
    clo = lax.dot_general(eflo_ref[...], w1e, dn_in,
                          preferred_element_type=jnp.float32)
    chi = lax.dot_general(efhi_ref[...], w1e, dn_in,
                          preferred_element_type=jnp.float32)
    hlo = jnp.maximum(clo + pres[:, :D_HID] + b1, 0.0)
    hhi = jnp.maximum(chi + pres[:, D_HID:] + b1, 0.0)
    olo_ref[...] = lax.dot_general(w2, hlo, dn_out,
                                   preferred_element_type=jnp.float32) + b2
    ohi_ref[...] = lax.dot_general(w2, hhi, dn_out,
                                   preferred_element_type=jnp.float32) + b2


_mlp_call = pl.pallas_call(
    _mlp_body,
    grid=(_N_BLKS,),
    in_specs=[
        pl.BlockSpec((D_EDGE, _PAIR_BLK), lambda i: (0, i)),
        pl.BlockSpec((D_EDGE, _PAIR_BLK), lambda i: (0, i + _N_BLKS)),
        pl.BlockSpec((_PAIR_BLK, 2 * D_HID), lambda i: (i, 0)),
        pl.BlockSpec((D_EDGE, D_HID), lambda i: (0, 0)),
        pl.BlockSpec((1, D_HID), lambda i: (0, 0)),
        pl.BlockSpec((D_HID, D_OUT), lambda i: (0, 0)),
        pl.BlockSpec((D_OUT, 1), lambda i: (0, 0)),
    ],
    out_specs=[
        pl.BlockSpec((D_OUT, _PAIR_BLK), lambda i: (0, i)),
        pl.BlockSpec((D_OUT, _PAIR_BLK), lambda i: (0, i)),
    ],
    out_shape=[
        jax.ShapeDtypeStruct((D_OUT, _HALF), jnp.float32),
        jax.ShapeDtypeStruct((D_OUT, _HALF), jnp.float32),
    ],
)


def kernel(edge_index, node_features, edge_features, W1, b1, W2, b2):
    src = edge_index[0].astype(jnp.int32)
    tgt = edge_index[1].astype(jnp.int32)
    w1e = W1[:D_EDGE]
    w1s = W1[D_EDGE:D_EDGE + D_NODE]
    w1t = W1[D_EDGE + D_NODE:]
    ts, tt = _proj_call(node_features, w1s, w1t)
    presum2 = _make_sc_gather()(src, tgt, ts, tt)
    eft = jnp.transpose(edge_features)
    out_lo, out_hi = _mlp_call(
        eft, eft, presum2, w1e, b1.reshape(1, D_HID), W2,
        b2.reshape(D_OUT, 1))
    outt = jnp.concatenate([out_lo, out_hi], axis=1)
    return jnp.transpose(outt)
